# baseline (speedup 1.0000x reference)
import functools

import jax
import jax.numpy as jnp
from jax import lax
from jax.experimental import pallas as pl
from jax.experimental.pallas import tpu as pltpu
from jax.experimental.pallas import tpu_sc as plsc

N = 10000
E = 320000
NC = 2
NS = 16
NW = NC * NS
EPT = E // NW
CH = 80
NCH = EPT // CH
NIT = (NCH - 1) // 4
NPAD = 10112
SLAB = NPAD // NS

_mesh = plsc.VectorSubcoreMesh(core_axis_name="c", subcore_axis_name="s")


def _zero_vmem_2d(buf, rows, cols):
    z = jnp.zeros((16,), jnp.float32)

    def body(r, _):
        for cc in range(cols // 16):
            buf[r, pl.ds(cc * 16, 16)] = z
        return 0

    lax.fori_loop(0, rows, body, 0)


def _zero_slab(buf, accs, rbase, rows_total, rows_buf):
    nfull, rem = divmod(rows_total, rows_buf)
    for k in range(nfull):
        pltpu.sync_copy(buf, accs.at[pl.ds(rbase + rows_buf * k, rows_buf)])
    if rem:
        pltpu.sync_copy(buf.at[pl.ds(0, rem)],
                        accs.at[pl.ds(rbase + rows_buf * nfull, rem)])


@functools.partial(
    pl.kernel,
    out_type=jax.ShapeDtypeStruct((NC * NPAD,), jnp.float32),
    mesh=_mesh,
    scratch_types=[
        pltpu.VMEM((CH,), jnp.int32),
        pltpu.VMEM((CH,), jnp.int32),
        pltpu.VMEM((CH,), jnp.int32),
        pltpu.VMEM((CH,), jnp.int32),
        pltpu.VMEM((CH,), jnp.float32),
        pltpu.VMEM((NPAD,), jnp.float32),
        pltpu.VMEM((NS * SLAB,), jnp.float32),
        pltpu.VMEM((SLAB,), jnp.float32),
        pltpu.VMEM_SHARED((NS * NPAD,), jnp.float32),
        pltpu.SemaphoreType.DMA,
        pltpu.SemaphoreType.DMA,
        pltpu.SemaphoreType.DMA,
        pltpu.SemaphoreType.DMA,
    ],
    compiler_params=pltpu.CompilerParams(use_tc_tiling_on_sc=False,
                                         needs_layout_passes=False),
)
def _sc_deg(ei_hbm, deg_hbm,
            d0, d1, d2, d3, ones_v, degloc, mtmp, zslab, stage,
            sd0, sd1, sd2, sd3):
    di = [d0, d1, d2, d3]
    ds = [sd0, sd1, sd2, sd3]
    c = lax.axis_index("c")
    s = lax.axis_index("s")
    ebase = (c * NS + s) * EPT
    one = jnp.ones((16,), jnp.float32)
    z = jnp.zeros((16,), jnp.float32)
    for i in range(CH // 16):
        ones_v[pl.ds(i * 16, 16)] = one

    def zbody(i, _):
        degloc[pl.ds(i * 16, 16)] = z
        return 0

    lax.fori_loop(0, NPAD // 16, zbody, 0)

    def _stage_idx(j, k):
        pltpu.async_copy(ei_hbm.at[pl.ds(E + ebase + k * CH, CH)], di[j], ds[j])

    def _wait_idx(j):
        pltpu.make_async_copy(ei_hbm.at[pl.ds(E + ebase, CH)], di[j], ds[j]).wait()

    for j in range(4):
        _stage_idx(j, j)

    ones16 = jnp.ones((16,), jnp.float32)

    def _count(j):
        for i in range(CH // 16):
            idx16 = di[j][pl.ds(i * 16, 16)]
            plsc.addupdate_scatter(degloc, [idx16], ones16)

    def step(it, _):
        k0 = it * 4
        for j in range(4):
            k = k0 + j
            _wait_idx(j)
            _count(j)

            @pl.when(k + 4 <= NCH - 1)
            def _():
                _stage_idx(j, k + 4)
        return 0

    lax.fori_loop(0, NIT, step, 0)
    _wait_idx(0)
    _count(0)
    pltpu.sync_copy(degloc, stage.at[pl.ds(s * NPAD, NPAD)])
    plsc.subcore_barrier()
    for t in range(NS):
        pltpu.sync_copy(stage.at[pl.ds(t * NPAD + s * SLAB, SLAB)],
                        mtmp.at[pl.ds(t * SLAB, SLAB)])

    def merge(w, _):
        off = w * 16
        acc = mtmp[pl.ds(off, 16)]
        for t in range(1, NS):
            acc = acc + mtmp[pl.ds(t * SLAB + off, 16)]
        zslab[pl.ds(off, 16)] = acc
        return 0

    lax.fori_loop(0, SLAB // 16, merge, 0)
    off = SLAB - 16
    acc = mtmp[pl.ds(off, 16)]
    for t in range(1, NS):
        acc = acc + mtmp[pl.ds(t * SLAB + off, 16)]
    zslab[pl.ds(off, 16)] = acc
    pltpu.sync_copy(zslab, deg_hbm.at[pl.ds(c * NPAD + s * SLAB, SLAB)])


def _make_sc_agg(D):
    @functools.partial(
        pl.kernel,
        out_type=jax.ShapeDtypeStruct((NC, NPAD, D), jnp.float32),
        mesh=_mesh,
        scratch_types=[
            pltpu.VMEM((CH,), jnp.int32),
            pltpu.VMEM((CH,), jnp.int32),
            pltpu.VMEM((CH,), jnp.int32),
            pltpu.VMEM((CH,), jnp.int32),
            pltpu.VMEM((CH,), jnp.int32),
            pltpu.VMEM((CH,), jnp.int32),
            pltpu.VMEM((CH,), jnp.int32),
            pltpu.VMEM((CH,), jnp.int32),
            pltpu.VMEM((CH, D), jnp.float32),
            pltpu.VMEM((CH, D), jnp.float32),
            pltpu.VMEM((CH, D), jnp.float32),
            pltpu.VMEM((CH, D), jnp.float32),
            pltpu.VMEM_SHARED((NPAD, D), jnp.float32),
            pltpu.SemaphoreType.DMA,
            pltpu.SemaphoreType.DMA,
            pltpu.SemaphoreType.DMA,
            pltpu.SemaphoreType.DMA,
            pltpu.SemaphoreType.DMA,
            pltpu.SemaphoreType.DMA,
            pltpu.SemaphoreType.DMA,
            pltpu.SemaphoreType.DMA,
            pltpu.SemaphoreType.DMA,
            pltpu.SemaphoreType.DMA,
            pltpu.SemaphoreType.DMA,
            pltpu.SemaphoreType.DMA,
            pltpu.SemaphoreType.DMA,
            pltpu.SemaphoreType.DMA,
            pltpu.SemaphoreType.DMA,
            pltpu.SemaphoreType.DMA,
        ],
        compiler_params=pltpu.CompilerParams(use_tc_tiling_on_sc=False),
    )
    def _sc_agg(h_hbm, ei_hbm, acc_hbm,
                s0, s1, s2, s3, d0, d1, d2, d3, b0, b1, b2, b3, accs,
                es0, es1, es2, es3, ds0, ds1, ds2, ds3,
                gs0, gs1, gs2, gs3, ss0, ss1, ss2, ss3):
        si = [s0, s1, s2, s3]
        di = [d0, d1, d2, d3]
        bufs = [b0, b1, b2, b3]
        es = [es0, es1, es2, es3]
        ds = [ds0, ds1, ds2, ds3]
        gs = [gs0, gs1, gs2, gs3]
        ss = [ss0, ss1, ss2, ss3]
        c = lax.axis_index("c")
        s = lax.axis_index("s")
        ebase = (c * NS + s) * EPT
        _zero_vmem_2d(b0, CH, D)
        rbase = s * SLAB
        _zero_slab(b0, accs, rbase, SLAB, CH)
        plsc.subcore_barrier()

        def _stage_src(j, k):
            pltpu.async_copy(ei_hbm.at[pl.ds(ebase + k * CH, CH)], si[j], es[j])

        def _wait_src(j):
            pltpu.make_async_copy(ei_hbm.at[pl.ds(ebase, CH)], si[j], es[j]).wait()

        def _stage_dst(j, k):
            pltpu.async_copy(ei_hbm.at[pl.ds(E + ebase + k * CH, CH)], di[j], ds[j])

        def _wait_dst(j):
            pltpu.make_async_copy(ei_hbm.at[pl.ds(E + ebase, CH)], di[j], ds[j]).wait()

        for j in range(4):
            _stage_src(j, j)
        _stage_dst(0, 0)
        _stage_dst(1, 1)
        _wait_src(0)
        pltpu.async_copy(h_hbm.at[si[0]], b0, gs[0])
        _wait_src(1)
        pltpu.async_copy(h_hbm.at[si[1]], b1, gs[1])

        def step(it, _):
            k0 = it * 4
            for j in range(4):
                k = k0 + j
                j2 = (j + 2) % 4
                pltpu.make_async_copy(h_hbm.at[si[j]], bufs[j], gs[j]).wait()

                @pl.when(k + 4 <= NCH - 1)
                def _():
                    _stage_src(j, k + 4)

                _wait_dst(j)
                pltpu.sync_copy(bufs[j], accs.at[di[j]], add=True)

                @pl.when(k + 2 <= NCH - 1)
                def _():
                    _stage_dst(j2, k + 2)
                    _wait_src(j2)
                    pltpu.async_copy(h_hbm.at[si[j2]], bufs[j2], gs[j2])
            return 0

        lax.fori_loop(0, NIT, step, 0)
        pltpu.make_async_copy(h_hbm.at[si[0]], b0, gs[0]).wait()
        _wait_dst(0)
        pltpu.sync_copy(b0, accs.at[di[0]], add=True)
        plsc.subcore_barrier()
        nfull, rem = divmod(SLAB, CH)
        for k in range(nfull + 1):
            rows = CH if k < nfull else rem
            pltpu.sync_copy(accs.at[pl.ds(rbase + CH * k, rows)],
                            b0.at[pl.ds(0, rows)])
            pltpu.sync_copy(b0.at[pl.ds(0, rows)],
                            acc_hbm.at[c, pl.ds(rbase + CH * k, rows)])

    return _sc_agg


_sc_agg128 = _make_sc_agg(128)
_sc_agg64 = _make_sc_agg(64)


def _tc_a(x_ref, w_ref, deg_ref, h1p_ref, dis_ref):
    deg = deg_ref[0] + deg_ref[1] + 1.0
    dis = lax.rsqrt(deg)
    dis_ref[...] = dis
    h = jnp.dot(x_ref[...], w_ref[...], preferred_element_type=jnp.float32)
    h1p_ref[...] = h * dis


def _tc_b(acc_ref, h1p_ref, dis_ref, b1_ref, w2_ref, h2p_ref):
    dis = dis_ref[...]
    tot = acc_ref[0] + acc_ref[1] + h1p_ref[...]
    a1 = jnp.maximum(tot * dis + b1_ref[...], 0.0)
    h2p_ref[...] = jnp.dot(a1, w2_ref[...], preferred_element_type=jnp.float32) * dis


def _tc_c(acc_ref, h2p_ref, dis_ref, b2_ref, out_ref):
    tot = acc_ref[0] + acc_ref[1] + h2p_ref[...]
    out_ref[...] = tot * dis_ref[...] + b2_ref[...]


@jax.jit
def kernel(x, edge_index, W1, b1, W2, b2):
    ei = edge_index.astype(jnp.int32).reshape(2 * E)

    degf = _sc_deg(ei)

    BA = N // 10
    h1p, dis = pl.pallas_call(
        _tc_a,
        grid=(10,),
        in_specs=[
            pl.BlockSpec((BA, 128), lambda g: (g, 0)),
            pl.BlockSpec((128, 128), lambda g: (0, 0)),
            pl.BlockSpec((NC, BA, 1), lambda g: (0, g, 0)),
        ],
        out_specs=[
            pl.BlockSpec((BA, 128), lambda g: (g, 0)),
            pl.BlockSpec((BA, 1), lambda g: (g, 0)),
        ],
        out_shape=[
            jax.ShapeDtypeStruct((NPAD, 128), jnp.float32),
            jax.ShapeDtypeStruct((NPAD, 1), jnp.float32),
        ],
    )(x, W1, degf.reshape(NC, NPAD, 1))

    acc1 = _sc_agg128(h1p, ei)

    BB = NPAD // 8
    h2p = pl.pallas_call(
        _tc_b,
        grid=(8,),
        in_specs=[
            pl.BlockSpec((NC, BB, 128), lambda g: (0, g, 0)),
            pl.BlockSpec((BB, 128), lambda g: (g, 0)),
            pl.BlockSpec((BB, 1), lambda g: (g, 0)),
            pl.BlockSpec((1, 128), lambda g: (0, 0)),
            pl.BlockSpec((128, 64), lambda g: (0, 0)),
        ],
        out_specs=pl.BlockSpec((BB, 64), lambda g: (g, 0)),
        out_shape=jax.ShapeDtypeStruct((NPAD, 64), jnp.float32),
    )(acc1, h1p, dis, b1.reshape(1, 128), W2)

    acc2 = _sc_agg64(h2p, ei)

    BC = N // 10
    out = pl.pallas_call(
        _tc_c,
        grid=(10,),
        in_specs=[
            pl.BlockSpec((NC, BC, 64), lambda g: (0, g, 0)),
            pl.BlockSpec((BC, 64), lambda g: (g, 0)),
            pl.BlockSpec((BC, 1), lambda g: (g, 0)),
            pl.BlockSpec((1, 64), lambda g: (0, 0)),
        ],
        out_specs=pl.BlockSpec((BC, 64), lambda g: (g, 0)),
        out_shape=jax.ShapeDtypeStruct((N, 64), jnp.float32),
    )(acc2, h2p, dis, b2.reshape(1, 64))

    return out


# speedup vs baseline: 1.0121x; 1.0121x over previous
import functools

import jax
import jax.numpy as jnp
from jax import lax
from jax.experimental import pallas as pl
from jax.experimental.pallas import tpu as pltpu
from jax.experimental.pallas import tpu_sc as plsc

N = 10000
E = 320000
NC = 2
NS = 16
NW = NC * NS
EPT = E // NW
CH = 80
NCH = EPT // CH
NIT = (NCH - 1) // 4
NPAD = 10112
SLAB = NPAD // NS

_mesh = plsc.VectorSubcoreMesh(core_axis_name="c", subcore_axis_name="s")


def _zero_vmem_2d(buf, rows, cols):
    z = jnp.zeros((16,), jnp.float32)

    def body(r, _):
        for cc in range(cols // 16):
            buf[r, pl.ds(cc * 16, 16)] = z
        return 0

    lax.fori_loop(0, rows, body, 0)


def _zero_slab(buf, accs, rbase, rows_total, rows_buf):
    nfull, rem = divmod(rows_total, rows_buf)
    for k in range(nfull):
        pltpu.sync_copy(buf, accs.at[pl.ds(rbase + rows_buf * k, rows_buf)])
    if rem:
        pltpu.sync_copy(buf.at[pl.ds(0, rem)],
                        accs.at[pl.ds(rbase + rows_buf * nfull, rem)])


@functools.partial(
    pl.kernel,
    out_type=jax.ShapeDtypeStruct((NC * NPAD,), jnp.float32),
    mesh=_mesh,
    scratch_types=[
        pltpu.VMEM((CH,), jnp.int32),
        pltpu.VMEM((CH,), jnp.int32),
        pltpu.VMEM((CH,), jnp.int32),
        pltpu.VMEM((CH,), jnp.int32),
        pltpu.VMEM((CH,), jnp.float32),
        pltpu.VMEM((NPAD,), jnp.float32),
        pltpu.VMEM((NS * SLAB,), jnp.float32),
        pltpu.VMEM((SLAB,), jnp.float32),
        pltpu.VMEM_SHARED((NS * NPAD,), jnp.float32),
        pltpu.SemaphoreType.DMA,
        pltpu.SemaphoreType.DMA,
        pltpu.SemaphoreType.DMA,
        pltpu.SemaphoreType.DMA,
    ],
    compiler_params=pltpu.CompilerParams(use_tc_tiling_on_sc=False,
                                         needs_layout_passes=False),
)
def _sc_deg(ei_hbm, deg_hbm,
            d0, d1, d2, d3, ones_v, degloc, mtmp, zslab, stage,
            sd0, sd1, sd2, sd3):
    di = [d0, d1, d2, d3]
    ds = [sd0, sd1, sd2, sd3]
    c = lax.axis_index("c")
    s = lax.axis_index("s")
    ebase = (c * NS + s) * EPT
    one = jnp.ones((16,), jnp.float32)
    z = jnp.zeros((16,), jnp.float32)
    for i in range(CH // 16):
        ones_v[pl.ds(i * 16, 16)] = one

    def zbody(i, _):
        degloc[pl.ds(i * 16, 16)] = z
        return 0

    lax.fori_loop(0, NPAD // 16, zbody, 0)

    def _stage_idx(j, k):
        pltpu.async_copy(ei_hbm.at[pl.ds(E + ebase + k * CH, CH)], di[j], ds[j])

    def _wait_idx(j):
        pltpu.make_async_copy(ei_hbm.at[pl.ds(E + ebase, CH)], di[j], ds[j]).wait()

    for j in range(4):
        _stage_idx(j, j)

    ones16 = jnp.ones((16,), jnp.float32)

    def _count(j):
        for i in range(CH // 16):
            idx16 = di[j][pl.ds(i * 16, 16)]
            plsc.addupdate_scatter(degloc, [idx16], ones16)

    def step(it, _):
        k0 = it * 4
        for j in range(4):
            k = k0 + j
            _wait_idx(j)
            _count(j)

            @pl.when(k + 4 <= NCH - 1)
            def _():
                _stage_idx(j, k + 4)
        return 0

    lax.fori_loop(0, NIT, step, 0)
    _wait_idx(0)
    _count(0)
    pltpu.sync_copy(degloc, stage.at[pl.ds(s * NPAD, NPAD)])
    plsc.subcore_barrier()
    for t in range(NS):
        pltpu.sync_copy(stage.at[pl.ds(t * NPAD + s * SLAB, SLAB)],
                        mtmp.at[pl.ds(t * SLAB, SLAB)])

    def merge(w, _):
        off = w * 16
        acc = mtmp[pl.ds(off, 16)]
        for t in range(1, NS):
            acc = acc + mtmp[pl.ds(t * SLAB + off, 16)]
        zslab[pl.ds(off, 16)] = acc
        return 0

    lax.fori_loop(0, SLAB // 16, merge, 0)
    off = SLAB - 16
    acc = mtmp[pl.ds(off, 16)]
    for t in range(1, NS):
        acc = acc + mtmp[pl.ds(t * SLAB + off, 16)]
    zslab[pl.ds(off, 16)] = acc
    pltpu.sync_copy(zslab, deg_hbm.at[pl.ds(c * NPAD + s * SLAB, SLAB)])


def _make_sc_agg(D):
    @functools.partial(
        pl.kernel,
        out_type=jax.ShapeDtypeStruct((NC, NPAD, D), jnp.float32),
        mesh=_mesh,
        scratch_types=[
            pltpu.VMEM((CH,), jnp.int32),
            pltpu.VMEM((CH,), jnp.int32),
            pltpu.VMEM((CH,), jnp.int32),
            pltpu.VMEM((CH,), jnp.int32),
            pltpu.VMEM((CH,), jnp.int32),
            pltpu.VMEM((CH,), jnp.int32),
            pltpu.VMEM((CH,), jnp.int32),
            pltpu.VMEM((CH,), jnp.int32),
            pltpu.VMEM((CH, D), jnp.float32),
            pltpu.VMEM((CH, D), jnp.float32),
            pltpu.VMEM((CH, D), jnp.float32),
            pltpu.VMEM((CH, D), jnp.float32),
            pltpu.VMEM_SHARED((NPAD, D), jnp.float32),
            pltpu.SemaphoreType.DMA,
            pltpu.SemaphoreType.DMA,
            pltpu.SemaphoreType.DMA,
            pltpu.SemaphoreType.DMA,
            pltpu.SemaphoreType.DMA,
            pltpu.SemaphoreType.DMA,
            pltpu.SemaphoreType.DMA,
            pltpu.SemaphoreType.DMA,
            pltpu.SemaphoreType.DMA,
            pltpu.SemaphoreType.DMA,
            pltpu.SemaphoreType.DMA,
            pltpu.SemaphoreType.DMA,
            pltpu.SemaphoreType.DMA,
            pltpu.SemaphoreType.DMA,
            pltpu.SemaphoreType.DMA,
            pltpu.SemaphoreType.DMA,
        ],
        compiler_params=pltpu.CompilerParams(use_tc_tiling_on_sc=False),
    )
    def _sc_agg(h_hbm, ei_hbm, acc_hbm,
                s0, s1, s2, s3, d0, d1, d2, d3, b0, b1, b2, b3, accs,
                es0, es1, es2, es3, ds0, ds1, ds2, ds3,
                gs0, gs1, gs2, gs3, ss0, ss1, ss2, ss3):
        si = [s0, s1, s2, s3]
        di = [d0, d1, d2, d3]
        bufs = [b0, b1, b2, b3]
        es = [es0, es1, es2, es3]
        ds = [ds0, ds1, ds2, ds3]
        gs = [gs0, gs1, gs2, gs3]
        ss = [ss0, ss1, ss2, ss3]
        c = lax.axis_index("c")
        s = lax.axis_index("s")
        ebase = (c * NS + s) * EPT
        _zero_vmem_2d(b0, CH, D)
        rbase = s * SLAB
        _zero_slab(b0, accs, rbase, SLAB, CH)
        plsc.subcore_barrier()

        def _stage_src(j, k):
            pltpu.async_copy(ei_hbm.at[pl.ds(ebase + k * CH, CH)], si[j], es[j])

        def _wait_src(j):
            pltpu.make_async_copy(ei_hbm.at[pl.ds(ebase, CH)], si[j], es[j]).wait()

        def _stage_dst(j, k):
            pltpu.async_copy(ei_hbm.at[pl.ds(E + ebase + k * CH, CH)], di[j], ds[j])

        def _wait_dst(j):
            pltpu.make_async_copy(ei_hbm.at[pl.ds(E + ebase, CH)], di[j], ds[j]).wait()

        for j in range(4):
            _stage_src(j, j)
        _stage_dst(0, 0)
        _stage_dst(1, 1)
        _wait_src(0)
        pltpu.async_copy(h_hbm.at[si[0]], b0, gs[0])
        _wait_src(1)
        pltpu.async_copy(h_hbm.at[si[1]], b1, gs[1])

        def step(it, _):
            k0 = it * 4
            for j in range(4):
                k = k0 + j
                j2 = (j + 2) % 4
                pltpu.make_async_copy(h_hbm.at[si[j]], bufs[j], gs[j]).wait()

                @pl.when(k + 4 <= NCH - 1)
                def _():
                    _stage_src(j, k + 4)

                _wait_dst(j)
                pltpu.sync_copy(bufs[j], accs.at[di[j]], add=True)

                @pl.when(k + 2 <= NCH - 1)
                def _():
                    _stage_dst(j2, k + 2)
                    _wait_src(j2)
                    pltpu.async_copy(h_hbm.at[si[j2]], bufs[j2], gs[j2])
            return 0

        lax.fori_loop(0, NIT, step, 0)
        pltpu.make_async_copy(h_hbm.at[si[0]], b0, gs[0]).wait()
        _wait_dst(0)
        pltpu.sync_copy(b0, accs.at[di[0]], add=True)
        plsc.subcore_barrier()
        nfull, rem = divmod(SLAB, CH)
        for k in range(nfull + 1):
            rows = CH if k < nfull else rem
            pltpu.sync_copy(accs.at[pl.ds(rbase + CH * k, rows)],
                            b0.at[pl.ds(0, rows)])
            pltpu.sync_copy(b0.at[pl.ds(0, rows)],
                            acc_hbm.at[c, pl.ds(rbase + CH * k, rows)])

    return _sc_agg


_sc_agg128 = _make_sc_agg(128)
_sc_agg64 = _make_sc_agg(64)


def _tc_a1(x_ref, w_ref, u_ref):
    u_ref[...] = jnp.dot(x_ref[...], w_ref[...],
                         preferred_element_type=jnp.float32)


def _tc_a2(u_ref, deg_ref, h1p_ref, dis_ref):
    deg = deg_ref[0] + deg_ref[1] + 1.0
    dis = lax.rsqrt(deg)
    dis_ref[...] = dis
    h1p_ref[pl.ds(0, N)] = u_ref[...] * dis[:N]
    h1p_ref[pl.ds(N, NPAD - N)] = jnp.zeros((NPAD - N, 128), jnp.float32)


def _tc_b(acc_ref, h1p_ref, dis_ref, b1_ref, w2_ref, h2p_ref):
    dis = dis_ref[...]
    tot = acc_ref[0] + acc_ref[1] + h1p_ref[...]
    a1 = jnp.maximum(tot * dis + b1_ref[...], 0.0)
    h2p_ref[...] = jnp.dot(a1, w2_ref[...], preferred_element_type=jnp.float32) * dis


def _tc_c(acc_ref, h2p_ref, dis_ref, b2_ref, out_ref):
    tot = acc_ref[0, pl.ds(0, N)] + acc_ref[1, pl.ds(0, N)] + h2p_ref[pl.ds(0, N)]
    out_ref[...] = tot * dis_ref[pl.ds(0, N)] + b2_ref[...]


@jax.jit
def kernel(x, edge_index, W1, b1, W2, b2):
    ei = edge_index.astype(jnp.int32).reshape(2 * E)

    degf = _sc_deg(ei)

    u1 = pl.pallas_call(
        _tc_a1,
        out_shape=jax.ShapeDtypeStruct((N, 128), jnp.float32),
    )(x, W1)

    h1p, dis = pl.pallas_call(
        _tc_a2,
        out_shape=[
            jax.ShapeDtypeStruct((NPAD, 128), jnp.float32),
            jax.ShapeDtypeStruct((NPAD, 1), jnp.float32),
        ],
    )(u1, degf.reshape(NC, NPAD, 1))

    acc1 = _sc_agg128(h1p, ei)

    h2p = pl.pallas_call(
        _tc_b,
        out_shape=jax.ShapeDtypeStruct((NPAD, 64), jnp.float32),
    )(acc1, h1p, dis, b1.reshape(1, 128), W2)

    acc2 = _sc_agg64(h2p, ei)

    out = pl.pallas_call(
        _tc_c,
        out_shape=jax.ShapeDtypeStruct((N, 64), jnp.float32),
    )(acc2, h2p, dis, b2.reshape(1, 64))

    return out


# speedup vs baseline: 1.1030x; 1.0899x over previous
import functools

import jax
import jax.numpy as jnp
from jax import lax
from jax.experimental import pallas as pl
from jax.experimental.pallas import tpu as pltpu
from jax.experimental.pallas import tpu_sc as plsc

N = 10000
E = 320000
NC = 2
NS = 16
NW = NC * NS
EPT = E // NW
CH = 80
NCH = EPT // CH
NIT = (NCH - 1) // 4
NPAD = 10112
SLAB = NPAD // NS

_mesh = plsc.VectorSubcoreMesh(core_axis_name="c", subcore_axis_name="s")


def _zero_vmem_2d(buf, rows, cols):
    z = jnp.zeros((16,), jnp.float32)

    def body(r, _):
        for cc in range(cols // 16):
            buf[r, pl.ds(cc * 16, 16)] = z
        return 0

    lax.fori_loop(0, rows, body, 0)


def _zero_slab(buf, accs, rbase, rows_total, rows_buf):
    nfull, rem = divmod(rows_total, rows_buf)
    for k in range(nfull):
        pltpu.sync_copy(buf, accs.at[pl.ds(rbase + rows_buf * k, rows_buf)])
    if rem:
        pltpu.sync_copy(buf.at[pl.ds(0, rem)],
                        accs.at[pl.ds(rbase + rows_buf * nfull, rem)])


@functools.partial(
    pl.kernel,
    out_type=jax.ShapeDtypeStruct((NC * NPAD,), jnp.float32),
    mesh=_mesh,
    scratch_types=[
        pltpu.VMEM((CH,), jnp.int32),
        pltpu.VMEM((CH,), jnp.int32),
        pltpu.VMEM((CH,), jnp.int32),
        pltpu.VMEM((CH,), jnp.int32),
        pltpu.VMEM((CH,), jnp.float32),
        pltpu.VMEM((NPAD,), jnp.float32),
        pltpu.VMEM((NS * SLAB,), jnp.float32),
        pltpu.VMEM((SLAB,), jnp.float32),
        pltpu.VMEM_SHARED((NS * NPAD,), jnp.float32),
        pltpu.SemaphoreType.DMA,
        pltpu.SemaphoreType.DMA,
        pltpu.SemaphoreType.DMA,
        pltpu.SemaphoreType.DMA,
    ],
    compiler_params=pltpu.CompilerParams(use_tc_tiling_on_sc=False,
                                         needs_layout_passes=False),
)
def _sc_deg(ei_hbm, deg_hbm,
            d0, d1, d2, d3, ones_v, degloc, mtmp, zslab, stage,
            sd0, sd1, sd2, sd3):
    di = [d0, d1, d2, d3]
    ds = [sd0, sd1, sd2, sd3]
    c = lax.axis_index("c")
    s = lax.axis_index("s")
    ebase = (c * NS + s) * EPT
    one = jnp.ones((16,), jnp.float32)
    z = jnp.zeros((16,), jnp.float32)
    for i in range(CH // 16):
        ones_v[pl.ds(i * 16, 16)] = one

    def zbody(i, _):
        degloc[pl.ds(i * 16, 16)] = z
        return 0

    lax.fori_loop(0, NPAD // 16, zbody, 0)

    def _stage_idx(j, k):
        pltpu.async_copy(ei_hbm.at[pl.ds(E + ebase + k * CH, CH)], di[j], ds[j])

    def _wait_idx(j):
        pltpu.make_async_copy(ei_hbm.at[pl.ds(E + ebase, CH)], di[j], ds[j]).wait()

    for j in range(4):
        _stage_idx(j, j)

    ones16 = jnp.ones((16,), jnp.float32)

    def _count(j):
        for i in range(CH // 16):
            idx16 = di[j][pl.ds(i * 16, 16)]
            plsc.addupdate_scatter(degloc, [idx16], ones16)

    def step(it, _):
        k0 = it * 4
        for j in range(4):
            k = k0 + j
            _wait_idx(j)
            _count(j)

            @pl.when(k + 4 <= NCH - 1)
            def _():
                _stage_idx(j, k + 4)
        return 0

    lax.fori_loop(0, NIT, step, 0)
    _wait_idx(0)
    _count(0)
    pltpu.sync_copy(degloc, stage.at[pl.ds(s * NPAD, NPAD)])
    plsc.subcore_barrier()
    for t in range(NS):
        pltpu.sync_copy(stage.at[pl.ds(t * NPAD + s * SLAB, SLAB)],
                        mtmp.at[pl.ds(t * SLAB, SLAB)])

    def merge(w, _):
        off = w * 16
        acc = mtmp[pl.ds(off, 16)]
        for t in range(1, NS):
            acc = acc + mtmp[pl.ds(t * SLAB + off, 16)]
        zslab[pl.ds(off, 16)] = acc
        return 0

    lax.fori_loop(0, SLAB // 16, merge, 0)
    off = SLAB - 16
    acc = mtmp[pl.ds(off, 16)]
    for t in range(1, NS):
        acc = acc + mtmp[pl.ds(t * SLAB + off, 16)]
    zslab[pl.ds(off, 16)] = acc
    pltpu.sync_copy(zslab, deg_hbm.at[pl.ds(c * NPAD + s * SLAB, SLAB)])


CHA = 128
NFULL = EPT // CHA
TAIL = EPT - NFULL * CHA


def _make_sc_agg(D):
    @functools.partial(
        pl.kernel,
        out_type=jax.ShapeDtypeStruct((NC, NPAD, D), jnp.float32),
        mesh=_mesh,
        scratch_types=[
            pltpu.VMEM((CHA,), jnp.int32),
            pltpu.VMEM((CHA,), jnp.int32),
            pltpu.VMEM((CHA,), jnp.int32),
            pltpu.VMEM((CHA,), jnp.int32),
            pltpu.VMEM((CHA,), jnp.int32),
            pltpu.VMEM((CHA,), jnp.int32),
            pltpu.VMEM((CHA,), jnp.int32),
            pltpu.VMEM((CHA,), jnp.int32),
            pltpu.VMEM((TAIL,), jnp.int32),
            pltpu.VMEM((TAIL,), jnp.int32),
            pltpu.VMEM((CHA, D), jnp.float32),
            pltpu.VMEM((CHA, D), jnp.float32),
            pltpu.VMEM_SHARED((NPAD, D), jnp.float32),
            pltpu.SemaphoreType.DMA,
            pltpu.SemaphoreType.DMA,
            pltpu.SemaphoreType.DMA,
            pltpu.SemaphoreType.DMA,
            pltpu.SemaphoreType.DMA,
            pltpu.SemaphoreType.DMA,
            pltpu.SemaphoreType.DMA,
            pltpu.SemaphoreType.DMA,
            pltpu.SemaphoreType.DMA,
            pltpu.SemaphoreType.DMA,
        ],
        compiler_params=pltpu.CompilerParams(use_tc_tiling_on_sc=False),
    )
    def _sc_agg(h_hbm, ei_hbm, acc_hbm,
                s0, s1, s2, s3, d0, d1, d2, d3, st, dt, b0, b1, accs,
                es0, es1, es2, es3, ds0, ds1, ds2, ds3, g0, g1):
        si = [s0, s1, s2, s3]
        di = [d0, d1, d2, d3]
        bufs = [b0, b1]
        es = [es0, es1, es2, es3]
        ds = [ds0, ds1, ds2, ds3]
        gs = [g0, g1]
        c = lax.axis_index("c")
        s = lax.axis_index("s")
        ebase = (c * NS + s) * EPT
        _zero_vmem_2d(b0, CHA, D)
        rbase = s * SLAB
        _zero_slab(b0, accs, rbase, SLAB, CHA)
        plsc.subcore_barrier()

        def _stage_src(j, k):
            pltpu.async_copy(ei_hbm.at[pl.ds(ebase + k * CHA, CHA)], si[j], es[j])

        def _wait_src(j):
            pltpu.make_async_copy(ei_hbm.at[pl.ds(ebase, CHA)], si[j], es[j]).wait()

        def _stage_dst(j, k):
            pltpu.async_copy(ei_hbm.at[pl.ds(E + ebase + k * CHA, CHA)], di[j], ds[j])

        def _wait_dst(j):
            pltpu.make_async_copy(ei_hbm.at[pl.ds(E + ebase, CHA)], di[j], ds[j]).wait()

        for j in range(4):
            _stage_src(j, j)
            _stage_dst(j, j)
        _wait_src(0)
        pltpu.async_copy(h_hbm.at[s0], b0, g0)
        _wait_src(1)
        pltpu.async_copy(h_hbm.at[s1], b1, g1)

        def step(it, _):
            k0 = it * 4
            for j in range(4):
                k = k0 + j
                p = j % 2
                j2 = (j + 2) % 4
                pltpu.make_async_copy(h_hbm.at[si[j]], bufs[p], gs[p]).wait()
                _wait_dst(j)
                pltpu.sync_copy(bufs[p], accs.at[di[j]], add=True)

                @pl.when(k + 4 <= NFULL - 1)
                def _():
                    _stage_src(j, k + 4)
                    _stage_dst(j, k + 4)

                @pl.when(k + 2 <= NFULL - 1)
                def _():
                    _wait_src(j2)
                    pltpu.async_copy(h_hbm.at[si[j2]], bufs[p], gs[p])
            return 0

        lax.fori_loop(0, NFULL // 4, step, 0)
        for j in range(2):
            pltpu.make_async_copy(h_hbm.at[si[j]], bufs[j], gs[j]).wait()
            _wait_dst(j)
            pltpu.sync_copy(bufs[j], accs.at[di[j]], add=True)
        pltpu.sync_copy(ei_hbm.at[pl.ds(ebase + NFULL * CHA, TAIL)], st)
        pltpu.sync_copy(ei_hbm.at[pl.ds(E + ebase + NFULL * CHA, TAIL)], dt)
        pltpu.async_copy(h_hbm.at[st], b0.at[pl.ds(0, TAIL)], g0).wait()
        pltpu.sync_copy(b0.at[pl.ds(0, TAIL)], accs.at[dt], add=True)
        plsc.subcore_barrier()
        nfull, rem = divmod(SLAB, CHA)
        for k in range(nfull + (1 if rem else 0)):
            rows = CHA if k < nfull else rem
            pltpu.sync_copy(accs.at[pl.ds(rbase + CHA * k, rows)],
                            b0.at[pl.ds(0, rows)])
            pltpu.sync_copy(b0.at[pl.ds(0, rows)],
                            acc_hbm.at[c, pl.ds(rbase + CHA * k, rows)])

    return _sc_agg


_sc_agg128 = _make_sc_agg(128)
_sc_agg64 = _make_sc_agg(64)


def _tc_a1(x_ref, w_ref, u_ref):
    u_ref[...] = jnp.dot(x_ref[...], w_ref[...],
                         preferred_element_type=jnp.float32)


def _tc_a2(u_ref, deg_ref, h1p_ref, dis_ref):
    deg = deg_ref[0] + deg_ref[1] + 1.0
    dis = lax.rsqrt(deg)
    dis_ref[...] = dis
    h1p_ref[pl.ds(0, N)] = u_ref[...] * dis[:N]
    h1p_ref[pl.ds(N, NPAD - N)] = jnp.zeros((NPAD - N, 128), jnp.float32)


def _tc_b(acc_ref, h1p_ref, dis_ref, b1_ref, w2_ref, h2p_ref):
    dis = dis_ref[...]
    tot = acc_ref[0] + acc_ref[1] + h1p_ref[...]
    a1 = jnp.maximum(tot * dis + b1_ref[...], 0.0)
    h2p_ref[...] = jnp.dot(a1, w2_ref[...], preferred_element_type=jnp.float32) * dis


def _tc_c(acc_ref, h2p_ref, dis_ref, b2_ref, out_ref):
    tot = acc_ref[0, pl.ds(0, N)] + acc_ref[1, pl.ds(0, N)] + h2p_ref[pl.ds(0, N)]
    out_ref[...] = tot * dis_ref[pl.ds(0, N)] + b2_ref[...]


@jax.jit
def kernel(x, edge_index, W1, b1, W2, b2):
    ei = edge_index.astype(jnp.int32).reshape(2 * E)

    degf = _sc_deg(ei)

    u1 = pl.pallas_call(
        _tc_a1,
        out_shape=jax.ShapeDtypeStruct((N, 128), jnp.float32),
    )(x, W1)

    h1p, dis = pl.pallas_call(
        _tc_a2,
        out_shape=[
            jax.ShapeDtypeStruct((NPAD, 128), jnp.float32),
            jax.ShapeDtypeStruct((NPAD, 1), jnp.float32),
        ],
    )(u1, degf.reshape(NC, NPAD, 1))

    acc1 = _sc_agg128(h1p, ei)

    h2p = pl.pallas_call(
        _tc_b,
        out_shape=jax.ShapeDtypeStruct((NPAD, 64), jnp.float32),
    )(acc1, h1p, dis, b1.reshape(1, 128), W2)

    acc2 = _sc_agg64(h2p, ei)

    out = pl.pallas_call(
        _tc_c,
        out_shape=jax.ShapeDtypeStruct((N, 64), jnp.float32),
    )(acc2, h2p, dis, b2.reshape(1, 64))

    return out
